# baseline (speedup 1.0000x reference)
import functools

import jax
import jax.numpy as jnp
from jax import lax
from jax.experimental import pallas as pl
from jax.experimental.pallas import tpu as pltpu
from jax.experimental.pallas import tpu_sc as plsc

N = 10000
E = 320000
D = 128
H = 128

NC = 2
NS = 16
C = 80

NP = 10240
RPT = NP // NS

PER_TILE = E // (NC * NS)
CHUNKS = PER_TILE // C

ZROWS = 40

GK = 3
NGROUPS = CHUNKS // GK

C1 = 128
K1_CHUNKS = PER_TILE // C1
K1_TAIL = PER_TILE - K1_CHUNKS * C1

BLK = 8
EP = ((E + BLK * 128 - 1) // (BLK * 128)) * (BLK * 128)
NB = EP // 128
NBLK = NB // BLK
NW = NC * NS
NDEG = N + 2000


@functools.cache
def _mesh():
    return plsc.VectorSubcoreMesh(
        core_axis_name="c", subcore_axis_name="s",
        num_cores=NC, num_subcores=NS)


def _fill_1d(ref, n, val):
    def body(i, _):
        ref[pl.ds(i * 16, 16)] = jnp.full((16,), val, jnp.float32)
        return 0
    lax.fori_loop(0, n // 16, body, 0)


def _fill_2d(ref, rows, cols, val):
    def body(i, _):
        def body2(j, _):
            ref[i, pl.ds(j * 16, 16)] = jnp.full((16,), val, jnp.float32)
            return 0
        lax.fori_loop(0, cols // 16, body2, 0)
        return 0
    lax.fori_loop(0, rows, body, 0)


@functools.cache
def _k1_build():
    @functools.partial(
        pl.kernel,
        mesh=_mesh(),
        out_type=[
            jax.ShapeDtypeStruct((NC * N,), jnp.float32),
            jax.ShapeDtypeStruct((NC * N,), jnp.float32),
        ],
        scratch_types=[
            pltpu.VMEM_SHARED((N,), jnp.float32),
            pltpu.VMEM_SHARED((N,), jnp.float32),
            pltpu.VMEM((2, C1), jnp.int32),
            pltpu.VMEM((2, C1), jnp.int32),
            pltpu.VMEM((C1,), jnp.float32),
            pltpu.VMEM((K1_TAIL,), jnp.int32),
            pltpu.VMEM((K1_TAIL,), jnp.float32),
            pltpu.VMEM((2000,), jnp.float32),
            pltpu.SemaphoreType.DMA,
        ],
    )
    def _k1(src_hbm, dst_hbm, do_out, di_out, do_sh, di_sh,
            si2, di2, ones_v, ti_v, tones_v, zb_v, lsem):
        c = lax.axis_index("c")
        s = lax.axis_index("s")

        _fill_1d(zb_v, 2000, 0.0)

        @pl.when(s < 5)
        def _():
            pltpu.sync_copy(zb_v, do_sh.at[pl.ds(s * 2000, 2000)])

        @pl.when(jnp.logical_and(s >= 5, s < 10))
        def _():
            pltpu.sync_copy(zb_v, di_sh.at[pl.ds((s - 5) * 2000, 2000)])

        _fill_1d(ones_v, C1, 1.0)
        _fill_1d(tones_v, K1_TAIL, 1.0)

        plsc.subcore_barrier()

        base0 = c * (E // NC) + s * PER_TILE

        pltpu.async_copy(src_hbm.at[pl.ds(base0, C1)], si2.at[0], lsem)
        pltpu.async_copy(dst_hbm.at[pl.ds(base0, C1)], di2.at[0], lsem)

        def body(i, _):
            p = lax.rem(i, 2)
            q = 1 - p

            @pl.when(i < K1_CHUNKS - 1)
            def _():
                base = base0 + (i + 1) * C1
                pltpu.async_copy(src_hbm.at[pl.ds(base, C1)], si2.at[q], lsem)
                pltpu.async_copy(dst_hbm.at[pl.ds(base, C1)], di2.at[q], lsem)

            base = base0 + i * C1
            pltpu.make_async_copy(
                src_hbm.at[pl.ds(base, C1)], si2.at[p], lsem).wait()
            pltpu.make_async_copy(
                dst_hbm.at[pl.ds(base, C1)], di2.at[p], lsem).wait()

            pltpu.sync_copy(ones_v, do_sh.at[si2.at[p]], add=True)
            pltpu.sync_copy(ones_v, di_sh.at[di2.at[p]], add=True)
            return 0
        lax.fori_loop(0, K1_CHUNKS, body, 0)

        base_t = base0 + K1_CHUNKS * C1
        pltpu.sync_copy(src_hbm.at[pl.ds(base_t, K1_TAIL)], ti_v)
        pltpu.sync_copy(tones_v, do_sh.at[ti_v], add=True)
        pltpu.sync_copy(dst_hbm.at[pl.ds(base_t, K1_TAIL)], ti_v)
        pltpu.sync_copy(tones_v, di_sh.at[ti_v], add=True)

        plsc.subcore_barrier()

        @pl.when(s < 5)
        def _():
            pltpu.sync_copy(do_sh.at[pl.ds(s * 2000, 2000)], zb_v)
            pltpu.sync_copy(zb_v, do_out.at[pl.ds(c * N + s * 2000, 2000)])

        @pl.when(jnp.logical_and(s >= 5, s < 10))
        def _():
            pltpu.sync_copy(di_sh.at[pl.ds((s - 5) * 2000, 2000)], zb_v)
            pltpu.sync_copy(
                zb_v, di_out.at[pl.ds(c * N + (s - 5) * 2000, 2000)])

    return _k1


def _k2a_body(x_ref, w_ref, xw_ref):
    xw_ref[...] = jnp.dot(x_ref[...], w_ref[...],
                          preferred_element_type=jnp.float32)


def _k2a(features, W):
    return pl.pallas_call(
        _k2a_body,
        out_shape=jax.ShapeDtypeStruct((N, H), jnp.float32),
    )(features, W)


def _k2b_body(xw_ref, dop_ref, h_ref):
    deg = dop_ref[0] + dop_ref[1]
    norm = jnp.where(deg > 0, lax.rsqrt(deg), 0.0)
    h_ref[...] = xw_ref[...] * norm[:, None]


def _k2b(xw, deg_o_part):
    return pl.pallas_call(
        _k2b_body,
        out_shape=jax.ShapeDtypeStruct((N, H), jnp.float32),
    )(xw, deg_o_part)


@functools.cache
def _k3_build():
    @functools.partial(
        pl.kernel,
        mesh=_mesh(),
        out_type=jax.ShapeDtypeStruct((NC, NP, H), jnp.float32),
        scratch_types=[
            pltpu.VMEM_SHARED((NP, H), jnp.float32),
            pltpu.VMEM((2 * GK, C), jnp.int32),
            pltpu.VMEM((2 * GK, C), jnp.int32),
            pltpu.VMEM((GK, C, H), jnp.float32),
            pltpu.VMEM((ZROWS, H), jnp.float32),
            pltpu.SemaphoreType.DMA,
            pltpu.SemaphoreType.DMA,
        ],
    )
    def _k3(h_hbm, src_hbm, dst_hbm, out_hbm,
            agg_sh, si6, di6, rows3, zb_v, lsem, gsem):
        c = lax.axis_index("c")
        s = lax.axis_index("s")

        _fill_2d(zb_v, ZROWS, H, 0.0)
        for r in range(RPT // ZROWS):
            pltpu.sync_copy(
                zb_v, agg_sh.at[pl.ds(s * RPT + r * ZROWS, ZROWS)])

        plsc.subcore_barrier()

        base0 = c * (E // NC) + s * PER_TILE

        def load_group_idx(j, slot0):
            for k in range(GK):
                base = base0 + (j * GK + k) * C
                pltpu.async_copy(src_hbm.at[pl.ds(base, C)],
                                 si6.at[slot0 + k], lsem)
                pltpu.async_copy(dst_hbm.at[pl.ds(base, C)],
                                 di6.at[slot0 + k], lsem)

        load_group_idx(0, 0)

        def body(j, _):
            pb = GK * lax.rem(j, 2)
            qb = GK - pb

            for k in range(GK):
                base = base0 + (j * GK + k) * C
                pltpu.make_async_copy(
                    src_hbm.at[pl.ds(base, C)], si6.at[pb + k], lsem).wait()
                pltpu.make_async_copy(
                    dst_hbm.at[pl.ds(base, C)], di6.at[pb + k], lsem).wait()

            handles = [
                pltpu.async_copy(h_hbm.at[si6.at[pb + k]], rows3.at[k], gsem)
                for k in range(GK)
            ]

            @pl.when(j < NGROUPS - 1)
            def _():
                def pf(k, _):
                    base = base0 + ((j + 1) * GK + k) * C
                    pltpu.async_copy(src_hbm.at[pl.ds(base, C)],
                                     si6.at[qb + k], lsem)
                    pltpu.async_copy(dst_hbm.at[pl.ds(base, C)],
                                     di6.at[qb + k], lsem)
                    return 0
                lax.fori_loop(0, GK, pf, 0)

            for k in range(GK):
                handles[k].wait()
                pltpu.sync_copy(rows3.at[k], agg_sh.at[di6.at[pb + k]],
                                add=True)
            return 0
        lax.fori_loop(0, NGROUPS, body, 0)

        for t in range(NGROUPS * GK, CHUNKS):
            base_t = base0 + t * C
            pltpu.sync_copy(src_hbm.at[pl.ds(base_t, C)], si6.at[0])
            gt = pltpu.async_copy(h_hbm.at[si6.at[0]], rows3.at[0], gsem)
            pltpu.sync_copy(dst_hbm.at[pl.ds(base_t, C)], di6.at[0])
            gt.wait()
            pltpu.sync_copy(rows3.at[0], agg_sh.at[di6.at[0]], add=True)

        plsc.subcore_barrier()

        for r in range(RPT // ZROWS):
            off = s * RPT + r * ZROWS
            pltpu.sync_copy(agg_sh.at[pl.ds(off, ZROWS)], zb_v)
            pltpu.sync_copy(zb_v, out_hbm.at[c, pl.ds(off, ZROWS)])

    return _k3


def _k4_body(agg_ref, dip_ref, b_ref, out_ref):
    deg = dip_ref[0] + dip_ref[1]
    norm = jnp.where(deg > 0, lax.rsqrt(deg), 0.0)
    agg = agg_ref[0, :N, :] + agg_ref[1, :N, :]
    out_ref[...] = jnp.maximum(agg * norm[:, None] + b_ref[...], 0.0)


def _k4(agg_split, deg_i_part, b):
    return pl.pallas_call(
        _k4_body,
        out_shape=jax.ShapeDtypeStruct((N, H), jnp.float32),
    )(agg_split, deg_i_part, b.reshape(1, H))


def kernel(features, edge_index, W, b):
    src = edge_index[0]
    dst = edge_index[1]
    xw = _k2a(features, W)
    deg_o_flat, deg_i_flat = _k1_build()(src, dst)
    deg_o_part = deg_o_flat.reshape(NC, N)
    deg_i_part = deg_i_flat.reshape(NC, N)
    h = _k2b(xw, deg_o_part)
    agg_split = _k3_build()(h, src, dst)
    return _k4(agg_split, deg_i_part, b)


# speedup vs baseline: 1.2013x; 1.2013x over previous
import functools

import jax
import jax.numpy as jnp
from jax import lax
from jax.experimental import pallas as pl
from jax.experimental.pallas import tpu as pltpu
from jax.experimental.pallas import tpu_sc as plsc

N = 10000
E = 320000
D = 128
H = 128

NC = 2
NS = 16
C = 80

NP = 10240
RPT = NP // NS

PER_TILE = E // (NC * NS)
CHUNKS = PER_TILE // C

ZROWS = 40

GK = 3
NGROUPS = CHUNKS // GK

C1 = 128
K1_CHUNKS = PER_TILE // C1
K1_TAIL = PER_TILE - K1_CHUNKS * C1

BLK = 8
EP = ((E + BLK * 128 - 1) // (BLK * 128)) * (BLK * 128)
NB = EP // 128
NBLK = NB // BLK
NW = NC * NS
NDEG = N + 2000


@functools.cache
def _mesh():
    return plsc.VectorSubcoreMesh(
        core_axis_name="c", subcore_axis_name="s",
        num_cores=NC, num_subcores=NS)


def _fill_1d(ref, n, val):
    def body(i, _):
        ref[pl.ds(i * 16, 16)] = jnp.full((16,), val, jnp.float32)
        return 0
    lax.fori_loop(0, n // 16, body, 0)


def _fill_2d(ref, rows, cols, val):
    def body(i, _):
        def body2(j, _):
            ref[i, pl.ds(j * 16, 16)] = jnp.full((16,), val, jnp.float32)
            return 0
        lax.fori_loop(0, cols // 16, body2, 0)
        return 0
    lax.fori_loop(0, rows, body, 0)


@functools.cache
def _k1_build():
    @functools.partial(
        pl.kernel,
        mesh=_mesh(),
        out_type=[
            jax.ShapeDtypeStruct((NC * N,), jnp.float32),
            jax.ShapeDtypeStruct((NC * N,), jnp.float32),
        ],
        scratch_types=[
            pltpu.VMEM_SHARED((N,), jnp.float32),
            pltpu.VMEM_SHARED((N,), jnp.float32),
            pltpu.VMEM((2, C1), jnp.int32),
            pltpu.VMEM((2, C1), jnp.int32),
            pltpu.VMEM((C1,), jnp.float32),
            pltpu.VMEM((K1_TAIL,), jnp.int32),
            pltpu.VMEM((K1_TAIL,), jnp.float32),
            pltpu.VMEM((2000,), jnp.float32),
            pltpu.SemaphoreType.DMA,
        ],
    )
    def _k1(src_hbm, dst_hbm, do_out, di_out, do_sh, di_sh,
            si2, di2, ones_v, ti_v, tones_v, zb_v, lsem):
        c = lax.axis_index("c")
        s = lax.axis_index("s")

        _fill_1d(zb_v, 2000, 0.0)

        @pl.when(s < 5)
        def _():
            pltpu.sync_copy(zb_v, do_sh.at[pl.ds(s * 2000, 2000)])

        @pl.when(jnp.logical_and(s >= 5, s < 10))
        def _():
            pltpu.sync_copy(zb_v, di_sh.at[pl.ds((s - 5) * 2000, 2000)])

        _fill_1d(ones_v, C1, 1.0)
        _fill_1d(tones_v, K1_TAIL, 1.0)

        plsc.subcore_barrier()

        base0 = c * (E // NC) + s * PER_TILE

        pltpu.async_copy(src_hbm.at[pl.ds(base0, C1)], si2.at[0], lsem)
        pltpu.async_copy(dst_hbm.at[pl.ds(base0, C1)], di2.at[0], lsem)

        def body(i, _):
            p = lax.rem(i, 2)
            q = 1 - p

            @pl.when(i < K1_CHUNKS - 1)
            def _():
                base = base0 + (i + 1) * C1
                pltpu.async_copy(src_hbm.at[pl.ds(base, C1)], si2.at[q], lsem)
                pltpu.async_copy(dst_hbm.at[pl.ds(base, C1)], di2.at[q], lsem)

            base = base0 + i * C1
            pltpu.make_async_copy(
                src_hbm.at[pl.ds(base, C1)], si2.at[p], lsem).wait()
            pltpu.make_async_copy(
                dst_hbm.at[pl.ds(base, C1)], di2.at[p], lsem).wait()

            pltpu.sync_copy(ones_v, do_sh.at[si2.at[p]], add=True)
            pltpu.sync_copy(ones_v, di_sh.at[di2.at[p]], add=True)
            return 0
        lax.fori_loop(0, K1_CHUNKS, body, 0)

        base_t = base0 + K1_CHUNKS * C1
        pltpu.sync_copy(src_hbm.at[pl.ds(base_t, K1_TAIL)], ti_v)
        pltpu.sync_copy(tones_v, do_sh.at[ti_v], add=True)
        pltpu.sync_copy(dst_hbm.at[pl.ds(base_t, K1_TAIL)], ti_v)
        pltpu.sync_copy(tones_v, di_sh.at[ti_v], add=True)

        plsc.subcore_barrier()

        @pl.when(s < 5)
        def _():
            pltpu.sync_copy(do_sh.at[pl.ds(s * 2000, 2000)], zb_v)
            pltpu.sync_copy(zb_v, do_out.at[pl.ds(c * N + s * 2000, 2000)])

        @pl.when(jnp.logical_and(s >= 5, s < 10))
        def _():
            pltpu.sync_copy(di_sh.at[pl.ds((s - 5) * 2000, 2000)], zb_v)
            pltpu.sync_copy(
                zb_v, di_out.at[pl.ds(c * N + (s - 5) * 2000, 2000)])

    return _k1


def _k2_body(x_ref, w_ref, dop_ref, h_ref):
    deg = dop_ref[0] + dop_ref[1]
    norm = jnp.where(deg > 0, lax.rsqrt(deg), 0.0)
    xw = jnp.dot(x_ref[...], w_ref[...], preferred_element_type=jnp.float32)
    h_ref[...] = xw * norm[:, None]


def _k2(features, W, deg_o_part):
    return pl.pallas_call(
        _k2_body,
        out_shape=jax.ShapeDtypeStruct((N, H), jnp.float32),
    )(features, W, deg_o_part)


@functools.cache
def _k3_build():
    @functools.partial(
        pl.kernel,
        mesh=_mesh(),
        out_type=jax.ShapeDtypeStruct((NC, NP, H), jnp.float32),
        scratch_types=[
            pltpu.VMEM_SHARED((NP, H), jnp.float32),
            pltpu.VMEM((4, C), jnp.int32),
            pltpu.VMEM((4, C), jnp.int32),
            pltpu.VMEM((3, C, H), jnp.float32),
            pltpu.VMEM((ZROWS, H), jnp.float32),
            pltpu.SemaphoreType.DMA,
            pltpu.SemaphoreType.DMA,
        ],
    )
    def _k3(h_hbm, src_hbm, dst_hbm, out_hbm,
            agg_sh, si4, di4, rows3, zb_v, lsem, gsem):
        c = lax.axis_index("c")
        s = lax.axis_index("s")

        _fill_2d(zb_v, ZROWS, H, 0.0)
        for r in range(RPT // ZROWS):
            pltpu.sync_copy(
                zb_v, agg_sh.at[pl.ds(s * RPT + r * ZROWS, ZROWS)])

        plsc.subcore_barrier()

        base0 = c * (E // NC) + s * PER_TILE

        def load_idx(i, slot):
            base = base0 + i * C
            pltpu.async_copy(src_hbm.at[pl.ds(base, C)], si4.at[slot], lsem)
            pltpu.async_copy(dst_hbm.at[pl.ds(base, C)], di4.at[slot], lsem)

        def drain_idx(i, slot):
            base = base0 + i * C
            pltpu.make_async_copy(
                src_hbm.at[pl.ds(base, C)], si4.at[slot], lsem).wait()
            pltpu.make_async_copy(
                dst_hbm.at[pl.ds(base, C)], di4.at[slot], lsem).wait()

        for i in range(3):
            load_idx(i, i)
        for i in range(2):
            drain_idx(i, i)
            pltpu.async_copy(h_hbm.at[si4.at[i]], rows3.at[i], gsem)

        def body(i, _):
            @pl.when(i + 2 < CHUNKS)
            def _():
                drain_idx(i + 2, lax.rem(i + 2, 4))
                pltpu.async_copy(h_hbm.at[si4.at[lax.rem(i + 2, 4)]],
                                 rows3.at[lax.rem(i + 2, 3)], gsem)

            @pl.when(i + 3 < CHUNKS)
            def _():
                load_idx(i + 3, lax.rem(i + 3, 4))

            p3 = lax.rem(i, 3)
            p4 = lax.rem(i, 4)
            pltpu.make_async_copy(
                h_hbm.at[si4.at[p4]], rows3.at[p3], gsem).wait()
            pltpu.sync_copy(rows3.at[p3], agg_sh.at[di4.at[p4]], add=True)
            return 0
        lax.fori_loop(0, CHUNKS, body, 0)

        plsc.subcore_barrier()

        for r in range(RPT // ZROWS):
            off = s * RPT + r * ZROWS
            pltpu.sync_copy(agg_sh.at[pl.ds(off, ZROWS)], zb_v)
            pltpu.sync_copy(zb_v, out_hbm.at[c, pl.ds(off, ZROWS)])

    return _k3


def _k4_body(agg_ref, dip_ref, b_ref, out_ref):
    deg = dip_ref[0] + dip_ref[1]
    norm = jnp.where(deg > 0, lax.rsqrt(deg), 0.0)
    agg = agg_ref[0, :N, :] + agg_ref[1, :N, :]
    out_ref[...] = jnp.maximum(agg * norm[:, None] + b_ref[...], 0.0)


def _k4(agg_split, deg_i_part, b):
    return pl.pallas_call(
        _k4_body,
        out_shape=jax.ShapeDtypeStruct((N, H), jnp.float32),
    )(agg_split, deg_i_part, b.reshape(1, H))


def kernel(features, edge_index, W, b):
    src = edge_index[0]
    dst = edge_index[1]
    deg_o_flat, deg_i_flat = _k1_build()(src, dst)
    deg_o_part = deg_o_flat.reshape(NC, N)
    deg_i_part = deg_i_flat.reshape(NC, N)
    h = _k2(features, W, deg_o_part)
    agg_split = _k3_build()(h, src, dst)
    return _k4(agg_split, deg_i_part, b)


# speedup vs baseline: 1.2313x; 1.0250x over previous
import functools

import jax
import jax.numpy as jnp
from jax import lax
from jax.experimental import pallas as pl
from jax.experimental.pallas import tpu as pltpu
from jax.experimental.pallas import tpu_sc as plsc

N = 10000
E = 320000
D = 128
H = 128

NC = 2
NS = 16
C = 80

NP = 10240
RPT = NP // NS

PER_TILE = E // (NC * NS)
CHUNKS = PER_TILE // C

ZROWS = 40

GK = 3
NGROUPS = CHUNKS // GK

C1 = 128
K1_CHUNKS = PER_TILE // C1
K1_TAIL = PER_TILE - K1_CHUNKS * C1

BLK = 8
EP = ((E + BLK * 128 - 1) // (BLK * 128)) * (BLK * 128)
NB = EP // 128
NBLK = NB // BLK
NW = NC * NS
NDEG = N + 2000


@functools.cache
def _mesh():
    return plsc.VectorSubcoreMesh(
        core_axis_name="c", subcore_axis_name="s",
        num_cores=NC, num_subcores=NS)


def _fill_1d(ref, n, val):
    def body(i, _):
        ref[pl.ds(i * 16, 16)] = jnp.full((16,), val, jnp.float32)
        return 0
    lax.fori_loop(0, n // 16, body, 0)


def _fill_2d(ref, rows, cols, val):
    def body(i, _):
        def body2(j, _):
            ref[i, pl.ds(j * 16, 16)] = jnp.full((16,), val, jnp.float32)
            return 0
        lax.fori_loop(0, cols // 16, body2, 0)
        return 0
    lax.fori_loop(0, rows, body, 0)


@functools.cache
def _k1_build():
    @functools.partial(
        pl.kernel,
        mesh=_mesh(),
        out_type=[
            jax.ShapeDtypeStruct((NC * N,), jnp.float32),
            jax.ShapeDtypeStruct((NC * N,), jnp.float32),
        ],
        scratch_types=[
            pltpu.VMEM_SHARED((N,), jnp.float32),
            pltpu.VMEM_SHARED((N,), jnp.float32),
            pltpu.VMEM((2, C1), jnp.int32),
            pltpu.VMEM((2, C1), jnp.int32),
            pltpu.VMEM((C1,), jnp.float32),
            pltpu.VMEM((K1_TAIL,), jnp.int32),
            pltpu.VMEM((K1_TAIL,), jnp.float32),
            pltpu.VMEM((2000,), jnp.float32),
            pltpu.SemaphoreType.DMA,
            pltpu.SemaphoreType.DMA,
        ],
    )
    def _k1(src_hbm, dst_hbm, do_out, di_out, do_sh, di_sh,
            si2, di2, ones_v, ti_v, tones_v, zb_v, lsem, asem):
        c = lax.axis_index("c")
        s = lax.axis_index("s")

        _fill_1d(zb_v, 2000, 0.0)

        @pl.when(s < 5)
        def _():
            pltpu.sync_copy(zb_v, do_sh.at[pl.ds(s * 2000, 2000)])

        @pl.when(jnp.logical_and(s >= 5, s < 10))
        def _():
            pltpu.sync_copy(zb_v, di_sh.at[pl.ds((s - 5) * 2000, 2000)])

        _fill_1d(ones_v, C1, 1.0)
        _fill_1d(tones_v, K1_TAIL, 1.0)

        plsc.subcore_barrier()

        base0 = c * (E // NC) + s * PER_TILE

        pltpu.async_copy(src_hbm.at[pl.ds(base0, C1)], si2.at[0], lsem)
        pltpu.async_copy(dst_hbm.at[pl.ds(base0, C1)], di2.at[0], lsem)

        def body(i, _):
            p = lax.rem(i, 2)
            q = 1 - p

            @pl.when(i > 0)
            def _():
                base_q = base0 + (i - 1) * C1
                pltpu.make_async_copy(
                    src_hbm.at[pl.ds(base_q, C1)], si2.at[q], asem).wait()
                pltpu.make_async_copy(
                    dst_hbm.at[pl.ds(base_q, C1)], di2.at[q], asem).wait()

            @pl.when(i < K1_CHUNKS - 1)
            def _():
                base = base0 + (i + 1) * C1
                pltpu.async_copy(src_hbm.at[pl.ds(base, C1)], si2.at[q], lsem)
                pltpu.async_copy(dst_hbm.at[pl.ds(base, C1)], di2.at[q], lsem)

            base = base0 + i * C1
            pltpu.make_async_copy(
                src_hbm.at[pl.ds(base, C1)], si2.at[p], lsem).wait()
            pltpu.make_async_copy(
                dst_hbm.at[pl.ds(base, C1)], di2.at[p], lsem).wait()

            pltpu.async_copy(ones_v, do_sh.at[si2.at[p]], asem, add=True)
            pltpu.async_copy(ones_v, di_sh.at[di2.at[p]], asem, add=True)
            return 0
        lax.fori_loop(0, K1_CHUNKS, body, 0)

        pfl = (K1_CHUNKS - 1) % 2
        base_l = base0 + (K1_CHUNKS - 1) * C1
        pltpu.make_async_copy(
            src_hbm.at[pl.ds(base_l, C1)], si2.at[pfl], asem).wait()
        pltpu.make_async_copy(
            dst_hbm.at[pl.ds(base_l, C1)], di2.at[pfl], asem).wait()

        base_t = base0 + K1_CHUNKS * C1
        pltpu.sync_copy(src_hbm.at[pl.ds(base_t, K1_TAIL)], ti_v)
        pltpu.sync_copy(tones_v, do_sh.at[ti_v], add=True)
        pltpu.sync_copy(dst_hbm.at[pl.ds(base_t, K1_TAIL)], ti_v)
        pltpu.sync_copy(tones_v, di_sh.at[ti_v], add=True)

        plsc.subcore_barrier()

        @pl.when(s < 5)
        def _():
            pltpu.sync_copy(do_sh.at[pl.ds(s * 2000, 2000)], zb_v)
            pltpu.sync_copy(zb_v, do_out.at[pl.ds(c * N + s * 2000, 2000)])

        @pl.when(jnp.logical_and(s >= 5, s < 10))
        def _():
            pltpu.sync_copy(di_sh.at[pl.ds((s - 5) * 2000, 2000)], zb_v)
            pltpu.sync_copy(
                zb_v, di_out.at[pl.ds(c * N + (s - 5) * 2000, 2000)])

    return _k1


def _k2_body(x_ref, w_ref, dop_ref, h_ref):
    deg = dop_ref[0] + dop_ref[1]
    norm = jnp.where(deg > 0, lax.rsqrt(deg), 0.0)
    xw = jnp.dot(x_ref[...], w_ref[...], preferred_element_type=jnp.float32)
    h_ref[...] = xw * norm[:, None]


def _k2(features, W, deg_o_part):
    return pl.pallas_call(
        _k2_body,
        out_shape=jax.ShapeDtypeStruct((N, H), jnp.float32),
    )(features, W, deg_o_part)


@functools.cache
def _k3_build():
    @functools.partial(
        pl.kernel,
        mesh=_mesh(),
        out_type=jax.ShapeDtypeStruct((NC, NP, H), jnp.float32),
        scratch_types=[
            pltpu.VMEM_SHARED((NP, H), jnp.float32),
            pltpu.VMEM((4, C), jnp.int32),
            pltpu.VMEM((4, C), jnp.int32),
            pltpu.VMEM((3, C, H), jnp.float32),
            pltpu.VMEM((ZROWS, H), jnp.float32),
            pltpu.SemaphoreType.DMA,
            pltpu.SemaphoreType.DMA,
        ],
    )
    def _k3(h_hbm, src_hbm, dst_hbm, out_hbm,
            agg_sh, si4, di4, rows3, zb_v, lsem, gsem):
        c = lax.axis_index("c")
        s = lax.axis_index("s")

        _fill_2d(zb_v, ZROWS, H, 0.0)
        for r in range(RPT // ZROWS):
            pltpu.sync_copy(
                zb_v, agg_sh.at[pl.ds(s * RPT + r * ZROWS, ZROWS)])

        plsc.subcore_barrier()

        base0 = c * (E // NC) + s * PER_TILE

        def load_idx(i, slot):
            base = base0 + i * C
            pltpu.async_copy(src_hbm.at[pl.ds(base, C)], si4.at[slot], lsem)
            pltpu.async_copy(dst_hbm.at[pl.ds(base, C)], di4.at[slot], lsem)

        def drain_idx(i, slot):
            base = base0 + i * C
            pltpu.make_async_copy(
                src_hbm.at[pl.ds(base, C)], si4.at[slot], lsem).wait()
            pltpu.make_async_copy(
                dst_hbm.at[pl.ds(base, C)], di4.at[slot], lsem).wait()

        for i in range(3):
            load_idx(i, i)
        for i in range(2):
            drain_idx(i, i)
            pltpu.async_copy(h_hbm.at[si4.at[i]], rows3.at[i], gsem)

        def body(i, _):
            @pl.when(i + 2 < CHUNKS)
            def _():
                drain_idx(i + 2, lax.rem(i + 2, 4))
                pltpu.async_copy(h_hbm.at[si4.at[lax.rem(i + 2, 4)]],
                                 rows3.at[lax.rem(i + 2, 3)], gsem)

            @pl.when(i + 3 < CHUNKS)
            def _():
                load_idx(i + 3, lax.rem(i + 3, 4))

            p3 = lax.rem(i, 3)
            p4 = lax.rem(i, 4)
            pltpu.make_async_copy(
                h_hbm.at[si4.at[p4]], rows3.at[p3], gsem).wait()
            pltpu.sync_copy(rows3.at[p3], agg_sh.at[di4.at[p4]], add=True)
            return 0
        lax.fori_loop(0, CHUNKS, body, 0)

        plsc.subcore_barrier()

        for r in range(RPT // ZROWS):
            off = s * RPT + r * ZROWS
            pltpu.sync_copy(agg_sh.at[pl.ds(off, ZROWS)], zb_v)
            pltpu.sync_copy(zb_v, out_hbm.at[c, pl.ds(off, ZROWS)])

    return _k3


def _k4_body(agg_ref, dip_ref, b_ref, out_ref):
    deg = dip_ref[0] + dip_ref[1]
    norm = jnp.where(deg > 0, lax.rsqrt(deg), 0.0)
    agg = agg_ref[0, :N, :] + agg_ref[1, :N, :]
    out_ref[...] = jnp.maximum(agg * norm[:, None] + b_ref[...], 0.0)


def _k4(agg_split, deg_i_part, b):
    return pl.pallas_call(
        _k4_body,
        out_shape=jax.ShapeDtypeStruct((N, H), jnp.float32),
    )(agg_split, deg_i_part, b.reshape(1, H))


def kernel(features, edge_index, W, b):
    src = edge_index[0]
    dst = edge_index[1]
    deg_o_flat, deg_i_flat = _k1_build()(src, dst)
    deg_o_part = deg_o_flat.reshape(NC, N)
    deg_i_part = deg_i_flat.reshape(NC, N)
    h = _k2(features, W, deg_o_part)
    agg_split = _k3_build()(h, src, dst)
    return _k4(agg_split, deg_i_part, b)
